# Initial kernel scaffold; baseline (speedup 1.0000x reference)
import jax
import jax.numpy as jnp
from jax.experimental import pallas as pl


def kernel(M, extended_sub_adj, original_adj_mask):
    raise NotImplementedError("write your pallas kernel here")


# speedup vs baseline: 16.5828x; 16.5828x over previous
import jax
import jax.numpy as jnp
from jax.experimental import pallas as pl
from jax.experimental.pallas import tpu as pltpu

_N = 8192
_BR = 256
_TOPK = 128


def _body(mrow_ref, mgrid_ref, mcol_ref, adj_ref, out_ref, scal_ref, drow_ref):
    i = pl.program_id(0)

    @pl.when(i == 0)
    def _scalars():
        mg = mgrid_ref[...]
        ag = jnp.abs(jnp.tanh(mg))
        ki = jax.lax.bitcast_convert_type(ag, jnp.int32)
        gidx = (jax.lax.broadcasted_iota(jnp.int32, (64, 128), 0) * 128
                + jax.lax.broadcasted_iota(jnp.int32, (64, 128), 1))
        ki = jnp.where(gidx == 0, jnp.int32(-1), ki)

        def bs_val(_, carry):
            lo, hi = carry
            mid = lo + (hi - lo) // 2
            geq = jnp.sum((ki >= mid).astype(jnp.int32)) >= _TOPK
            return (jnp.where(geq, mid, lo), jnp.where(geq, hi, mid))

        lo, hi = jax.lax.fori_loop(
            0, 31, bs_val, (jnp.int32(-2), jnp.int32(0x3F800001)))
        t_bits = lo
        n_above = jnp.sum((ki > t_bits).astype(jnp.int32))
        r = _TOPK - n_above

        def bs_idx(_, carry):
            lo2, hi2 = carry
            mid = lo2 + (hi2 - lo2) // 2
            enough = jnp.sum(((ki == t_bits) & (gidx <= mid)).astype(jnp.int32)) >= r
            return (jnp.where(enough, lo2, mid), jnp.where(enough, mid, hi2))

        lo2, hi2 = jax.lax.fori_loop(
            0, 13, bs_idx, (jnp.int32(-1), jnp.int32(_N - 1)))
        scal_ref[0] = t_bits
        scal_ref[1] = hi2

        mr = mrow_ref[...]
        tr = jnp.tanh(mr)
        kir = jax.lax.bitcast_convert_type(jnp.abs(tr), jnp.int32)
        lidx = jax.lax.broadcasted_iota(jnp.int32, (1, _N), 1)
        kir = jnp.where(lidx == 0, jnp.int32(-1), kir)
        inset = (kir > t_bits) | ((kir == t_bits) & (lidx <= hi2))
        drow_ref[...] = jnp.where(
            inset & (tr > 0.5), 1.0,
            jnp.where(inset & (tr < -0.5), -1.0, 0.0))

    out_ref[...] = adj_ref[...]

    t_bits = scal_ref[0]
    j_bound = scal_ref[1]
    mc = mcol_ref[...]
    tc = jnp.tanh(mc)
    kic = jax.lax.bitcast_convert_type(jnp.abs(tc), jnp.int32)
    ridx = jax.lax.broadcasted_iota(jnp.int32, (_BR, 1), 0) + i * _BR
    kic = jnp.where(ridx == 0, jnp.int32(-1), kic)
    insetc = (kic > t_bits) | ((kic == t_bits) & (ridx <= j_bound))
    pos_c = insetc & (tc > 0.5)
    neg_c = insetc & (tc < -0.5)
    out_ref[:, 0:1] = jnp.where(
        pos_c, 1.0, jnp.where(neg_c, 0.0, adj_ref[:, 0:1]))

    @pl.when(i == 0)
    def _row_edit():
        dr = drow_ref[...]
        out_ref[0:1, :] = jnp.where(
            dr > 0.5, 1.0, jnp.where(dr < -0.5, 0.0, adj_ref[0:1, :]))


def kernel(M, extended_sub_adj, original_adj_mask):
    del original_adj_mask
    n = extended_sub_adj.shape[0]
    mfull = jnp.concatenate([jnp.zeros((1,), jnp.float32), M])
    mrow = mfull.reshape(1, n)
    mgrid = mfull.reshape(64, n // 64)
    mcol = mfull.reshape(n, 1)
    return pl.pallas_call(
        _body,
        grid=(n // _BR,),
        in_specs=[
            pl.BlockSpec((1, n), lambda i: (0, 0)),
            pl.BlockSpec((64, n // 64), lambda i: (0, 0)),
            pl.BlockSpec((_BR, 1), lambda i: (i, 0)),
            pl.BlockSpec((_BR, n), lambda i: (i, 0)),
        ],
        out_specs=pl.BlockSpec((_BR, n), lambda i: (i, 0)),
        out_shape=jax.ShapeDtypeStruct((n, n), jnp.float32),
        scratch_shapes=[
            pltpu.SMEM((2,), jnp.int32),
            pltpu.VMEM((1, n), jnp.float32),
        ],
        compiler_params=pltpu.CompilerParams(
            dimension_semantics=("arbitrary",),
        ),
    )(mrow, mgrid, mcol, extended_sub_adj)
